# Initial kernel scaffold; baseline (speedup 1.0000x reference)
import jax
import jax.numpy as jnp
from jax.experimental import pallas as pl


def kernel(input_ids, token_type_ids, visual_embeds, visual_token_type_ids, word_emb, position_emb, token_type_emb, vis_token_type_emb, vis_position_emb, W_proj, b_proj, ln_gamma, ln_beta):
    raise NotImplementedError("write your pallas kernel here")


# speedup vs baseline: 3.4457x; 3.4457x over previous
import functools

import jax
import jax.numpy as jnp
from jax import lax
from jax.experimental import pallas as pl
from jax.experimental.pallas import tpu as pltpu
from jax.experimental.pallas import tpu_sc as plsc

_B = 1024
_S = 200
_H = 128
_VREG = 36
_VD = 2048
_T = _S + _VREG
_EPS = 1e-12

_NC = 2
_NS = 16
_NW = _NC * _NS
_TOK = _B * _S
_BPW = _TOK // _NW
_CH = 128
_NCH = _BPW // _CH


def _make_sc_gather():
    mesh = plsc.VectorSubcoreMesh(
        core_axis_name="c", subcore_axis_name="s",
        num_cores=_NC, num_subcores=_NS)

    @functools.partial(
        pl.kernel,
        out_type=jax.ShapeDtypeStruct((_TOK, _H), jnp.float32),
        mesh=mesh,
        scratch_types=[
            pltpu.VMEM((_NCH, _CH), jnp.int32),
            pltpu.VMEM((_CH, _H), jnp.float32),
            pltpu.SemaphoreType.DMA,
        ],
    )
    def sc_gather(idx_hbm, table_hbm, out_hbm, idx_v, rows_v, sem):
        wid = lax.axis_index("s") * _NC + lax.axis_index("c")
        pltpu.sync_copy(idx_hbm.at[wid], idx_v)
        base = wid * _BPW

        def body(j, carry):
            pltpu.async_copy(table_hbm.at[idx_v.at[j]], rows_v, sem).wait()
            off = pl.multiple_of(base + j * _CH, _CH)
            pltpu.sync_copy(rows_v, out_hbm.at[pl.ds(off, _CH)])
            return carry

        lax.fori_loop(0, _NCH, body, 0)

    return sc_gather


_SC_GATHER_CACHE = []


def _sc_gather(ids2, table):
    if not _SC_GATHER_CACHE:
        _SC_GATHER_CACHE.append(_make_sc_gather())
    return _SC_GATHER_CACHE[0](ids2, table)

_BB = 8


def _tc_body(g_ref, tt_ref, vis_ref, vt_ref, w_ref, pos_ref, tte_ref,
             vtte_ref, vb_ref, gam_ref, bet_ref, out_ref):
    gam = gam_ref[0]
    bet = bet_ref[0]

    def ln(x):
        mu = jnp.mean(x, axis=-1, keepdims=True)
        xc = x - mu
        var = jnp.mean(xc * xc, axis=-1, keepdims=True)
        return xc * lax.rsqrt(var + _EPS) * gam + bet

    t0 = tte_ref[0]
    dt = tte_ref[1] - tte_ref[0]
    tt = tt_ref[...].astype(jnp.float32)[..., None]
    xt = g_ref[...] + pos_ref[...][None] + t0 + tt * dt
    out_ref[:, 0:_S, :] = ln(xt)

    v = vis_ref[...]
    vp = lax.dot_general(
        v.reshape(_BB * _VREG, _VD), w_ref[...],
        (((1,), (0,)), ((), ())), preferred_element_type=jnp.float32)
    vp = vp.reshape(_BB, _VREG, _H)
    vt0 = vtte_ref[0]
    vdt = vtte_ref[1] - vtte_ref[0]
    vt = vt_ref[...].astype(jnp.float32)[..., None]
    xv = vp + vb_ref[0] + vt0 + vt * vdt
    out_ref[:, _S:, :] = ln(xv)


def _tc_call(g3, token_type_ids, visual_embeds, visual_token_type_ids,
             w_proj, pos_text, tte, vtte, vb, gam, bet):
    grid = (_B // _BB,)
    return pl.pallas_call(
        _tc_body,
        grid=grid,
        in_specs=[
            pl.BlockSpec((_BB, _S, _H), lambda b: (b, 0, 0)),
            pl.BlockSpec((_BB, _S), lambda b: (b, 0)),
            pl.BlockSpec((_BB, _VREG, _VD), lambda b: (b, 0, 0)),
            pl.BlockSpec((_BB, _VREG), lambda b: (b, 0)),
            pl.BlockSpec((_VD, _H), lambda b: (0, 0)),
            pl.BlockSpec((_S, _H), lambda b: (0, 0)),
            pl.BlockSpec((2, _H), lambda b: (0, 0)),
            pl.BlockSpec((2, _H), lambda b: (0, 0)),
            pl.BlockSpec((1, _H), lambda b: (0, 0)),
            pl.BlockSpec((1, _H), lambda b: (0, 0)),
            pl.BlockSpec((1, _H), lambda b: (0, 0)),
        ],
        out_specs=pl.BlockSpec((_BB, _T, _H), lambda b: (b, 0, 0)),
        out_shape=jax.ShapeDtypeStruct((_B, _T, _H), jnp.float32),
    )(g3, token_type_ids, visual_embeds, visual_token_type_ids,
      w_proj, pos_text, tte, vtte, vb, gam, bet)


def kernel(input_ids, token_type_ids, visual_embeds, visual_token_type_ids,
           word_emb, position_emb, token_type_emb, vis_token_type_emb,
           vis_position_emb, W_proj, b_proj, ln_gamma, ln_beta):
    ids2 = input_ids.reshape(_NW, _NCH, _CH)
    g = _sc_gather(ids2, word_emb)
    g3 = g.reshape(_B, _S, _H)

    pos_text = position_emb[:_S]
    vb = (vis_position_emb[0] + b_proj).reshape(1, _H)
    gam = ln_gamma.reshape(1, _H)
    bet = ln_beta.reshape(1, _H)
    return _tc_call(g3, token_type_ids, visual_embeds, visual_token_type_ids,
                    W_proj, pos_text, token_type_emb, vis_token_type_emb,
                    vb, gam, bet)
